# baseline (speedup 1.0000x reference)
import functools

import jax
import jax.numpy as jnp
from jax import lax
from jax.experimental import pallas as pl
from jax.experimental.pallas import tpu as pltpu
from jax.experimental.pallas import tpu_sc as plsc

B = 16384
D = 64
L = 16
NC, NS = 2, 16
NW = NC * NS
BPW = B // NW
G = 16
NG = BPW // G
CH = 128
NCH = BPW // CH

_mesh = plsc.VectorSubcoreMesh(core_axis_name="c", subcore_axis_name="s")


def _tec_body(ids_hbm, table_hbm, out_hbm, idx_v, rows_v, wsq_v, rs_v, sem):
    wid = lax.axis_index("s") * NC + lax.axis_index("c")
    base = wid * BPW

    for c in range(NCH):
        pltpu.sync_copy(ids_hbm.at[wid * NCH + c], idx_v.at[c])
    copies = [
        pltpu.async_copy(
            table_hbm.at[idx_v.at[c]],
            rows_v.at[pl.ds(c * CH, CH)],
            sem,
        )
        for c in range(NCH)
    ]
    for cp in copies:
        cp.wait()

    iota = lax.iota(jnp.int32, L)

    def group(g, carry):
        r0 = g * G
        for i in range(G):
            w = None
            for q in range(D // L):
                v = rows_v[r0 + i, pl.ds(q * L, L)]
                sq = v * v
                w = sq if w is None else w + sq
            wsq_v[pl.ds(i * L, L)] = w
        acc = None
        for j in range(G):
            col = plsc.load_gather(wsq_v, [iota * L + j])
            acc = col if acc is None else acc + col
        s = jnp.maximum(acc, jnp.float32(1e-24))
        half = s * jnp.float32(0.5)
        yi = jnp.int32(0x5F3759DF) - lax.shift_right_logical(
            plsc.bitcast(s, jnp.int32), 1)
        y = plsc.bitcast(yi, jnp.float32)
        for _ in range(3):
            y = y * (jnp.float32(1.5) - half * y * y)
        rs_v[pl.ds(L, L)] = y
        for i in range(G):
            ri = plsc.load_gather(rs_v, [jnp.full((L,), L + i, jnp.int32)])
            for q in range(D // L):
                rows_v[r0 + i, pl.ds(q * L, L)] = (
                    rows_v[r0 + i, pl.ds(q * L, L)] * ri)
        return carry

    lax.fori_loop(0, NG, group, 0)
    pltpu.sync_copy(rows_v, out_hbm.at[pl.ds(base, BPW)])


@functools.partial(
    pl.kernel,
    out_type=jax.ShapeDtypeStruct((B, D), jnp.float32),
    mesh=_mesh,
    compiler_params=pltpu.CompilerParams(
        needs_layout_passes=False, use_tc_tiling_on_sc=False),
    scratch_types=[
        pltpu.VMEM((NCH, CH), jnp.int32),
        pltpu.VMEM((BPW, D), jnp.float32),
        pltpu.VMEM((G * L,), jnp.float32),
        pltpu.VMEM((2 * L,), jnp.float32),
        pltpu.SemaphoreType.DMA,
    ],
)
def _embed_norm(ids_hbm, table_hbm, out_hbm, idx_v, rows_v, wsq_v, rs_v, sem):
    _tec_body(ids_hbm, table_hbm, out_hbm, idx_v, rows_v, wsq_v, rs_v, sem)


def kernel(task_ids, table):
    ids2 = task_ids.astype(jnp.int32).reshape(B // CH, CH)
    return _embed_norm(ids2, table)
